# Initial kernel scaffold; baseline (speedup 1.0000x reference)
import jax
import jax.numpy as jnp
from jax.experimental import pallas as pl


def kernel(x, spa_w, qkv_w, proj_w, proj_b, dw1_w, dw1_b, dw2_w, dw2_b):
    raise NotImplementedError("write your pallas kernel here")


# speedup vs baseline: 1.1963x; 1.1963x over previous
import jax
import jax.numpy as jnp
import numpy as np
from jax.experimental import pallas as pl
from jax.experimental.pallas import tpu as pltpu
from jax.experimental.pallas import tpu_sc as plsc


def _a_kern(xf_ref, wt_ref, qkv_ref):
    qkv_ref[...] = jnp.dot(xf_ref[...], wt_ref[...],
                           preferred_element_type=jnp.float32)


def _run_a(xf, qkv_wt):
    return pl.pallas_call(
        _a_kern,
        out_shape=jax.ShapeDtypeStruct((4096, 768), jnp.float32),
    )(xf, qkv_wt)


def _s_kern(s_ref, u_ref, l_ref, idx_ref):
    sb = jax.lax.bitcast_convert_type(s_ref[...], jnp.int32)

    def step(_, carry):
        lo, hi = carry
        mid = lo + (hi - lo + 1) // 2
        cnt = jnp.sum((sb >= mid).astype(jnp.int32))
        return jnp.where(cnt >= 512, mid, lo), jnp.where(cnt >= 512, hi, mid - 1)

    T, _ = jax.lax.fori_loop(0, 31, step, (jnp.int32(0), jnp.int32(2**30 + 2**29)))

    U = u_ref[...]
    L = l_ref[...]

    def prefix_incl(x01):
        rp = jnp.dot(x01, U, preferred_element_type=jnp.float32,
                     precision=jax.lax.Precision.HIGHEST)
        bo = jnp.dot(L, rp[:, 127:128], preferred_element_type=jnp.float32,
                     precision=jax.lax.Precision.HIGHEST)
        return rp + bo

    gt01 = jnp.where(sb > T, 1.0, 0.0)
    eq01 = jnp.where(sb == T, 1.0, 0.0)
    quota = 512.0 - jnp.sum(gt01)
    pe = prefix_incl(eq01)
    sel01 = gt01 + jnp.where(pe <= quota, eq01, 0.0)
    ps = prefix_incl(sel01)
    rank = ps - sel01
    rank = jnp.where(sel01 > 0.5, rank, 512.0)

    r_col = jax.lax.broadcasted_iota(jnp.int32, (512, 1), 0).astype(jnp.float32)
    acc = jnp.zeros((512, 1), jnp.float32)
    for row in range(32):
        ind = jnp.where(rank[row:row + 1, :] == r_col, 1.0, 0.0)
        n_col = jax.lax.broadcasted_iota(jnp.int32, (128, 1), 0).astype(jnp.float32) + (row * 128.0)
        acc = acc + jnp.dot(ind, n_col, preferred_element_type=jnp.float32,
                            precision=jax.lax.Precision.HIGHEST)
    idx_ref[...] = acc.astype(jnp.int32)


def _run_s(scores2d, U, L):
    return pl.pallas_call(
        _s_kern,
        out_shape=jax.ShapeDtypeStruct((512, 1), jnp.int32),
    )(scores2d, U, L)


def _sc_body(idx_hbm, kv_hbm, out_hbm, idx_v, kv_buf):
    c = jax.lax.axis_index("c")
    s = jax.lax.axis_index("s")

    @pl.when((c == 0) & (s == 0))
    def _():
        pltpu.sync_copy(idx_hbm, idx_v)
        for j in range(4):
            pltpu.sync_copy(kv_hbm.at[idx_v.at[pl.ds(j * 128, 128)]], kv_buf)
            pltpu.sync_copy(kv_buf, out_hbm.at[pl.ds(j * 128, 128), :])


def _run_sc(idx, kv):
    f = pl.kernel(
        _sc_body,
        out_type=jax.ShapeDtypeStruct((512, 512), jnp.float32),
        mesh=plsc.VectorSubcoreMesh(core_axis_name="c", subcore_axis_name="s"),
        scratch_types=[
            pltpu.VMEM((512,), jnp.int32),
            pltpu.VMEM((128, 512), jnp.float32),
        ],
    )
    return f(idx, kv)


def _c_kern(q_ref, k_ref, v_ref, o_ref):
    q = q_ref[0]
    k = k_ref[0]
    v = v_ref[0]
    s = jax.lax.dot_general(q, k, (((1,), (1,)), ((), ())),
                            preferred_element_type=jnp.float32)
    s = s * (8.0 ** -0.5)
    mx = jnp.max(s, axis=1, keepdims=True)
    e = jnp.exp(s - mx)
    a = e / jnp.sum(e, axis=1, keepdims=True)
    o_ref[0] = jnp.dot(a, v, preferred_element_type=jnp.float32)


def _run_c(qh, kh, vh):
    return pl.pallas_call(
        _c_kern,
        grid=(32,),
        in_specs=[
            pl.BlockSpec((1, 4096, 8), lambda h: (h, 0, 0)),
            pl.BlockSpec((1, 512, 8), lambda h: (h, 0, 0)),
            pl.BlockSpec((1, 512, 8), lambda h: (h, 0, 0)),
        ],
        out_specs=pl.BlockSpec((1, 4096, 8), lambda h: (h, 0, 0)),
        out_shape=jax.ShapeDtypeStruct((32, 4096, 8), jnp.float32),
    )(qh, kh, vh)


def _d_kern(vp_ref, w_ref, b_ref, out_ref):
    acc = jnp.zeros((16, 16, 16, 256), jnp.float32)
    for dz in range(3):
        for dy in range(3):
            for dx in range(3):
                i = dz * 9 + dy * 3 + dx
                acc = acc + vp_ref[dz:dz + 16, dy:dy + 16, dx:dx + 16, :] * w_ref[i]
    out_ref[...] = acc + b_ref[0]


def _run_d(vpad, w27, b):
    return pl.pallas_call(
        _d_kern,
        out_shape=jax.ShapeDtypeStruct((16, 16, 16, 256), jnp.float32),
    )(vpad, w27, b)


def _e_kern(att_ref, dw_ref, pw_ref, cw_ref, b_ref, out_ref):
    out = jnp.dot(att_ref[...], pw_ref[...], preferred_element_type=jnp.float32)
    out = out + jnp.dot(dw_ref[...], cw_ref[...], preferred_element_type=jnp.float32)
    out_ref[...] = out + b_ref[0]


def _run_e(att, dwout, proj_wt, dw2_wt, bias):
    return pl.pallas_call(
        _e_kern,
        out_shape=jax.ShapeDtypeStruct((4096, 256), jnp.float32),
    )(att, dwout, proj_wt, dw2_wt, bias)


def kernel(x, spa_w, qkv_w, proj_w, proj_b, dw1_w, dw1_b, dw2_w, dw2_b):
    B, C, D, H, W = x.shape
    N = D * H * W

    avg_out = jnp.mean(x, axis=1, keepdims=True)
    max_out = jnp.max(x, axis=1, keepdims=True)
    sa = jax.lax.conv_general_dilated(
        jnp.concatenate([avg_out, max_out], axis=1), spa_w,
        window_strides=(1, 1, 1), padding=[(3, 3)] * 3,
        dimension_numbers=('NCDHW', 'OIDHW', 'NCDHW'))
    scores = jax.nn.sigmoid(sa).reshape(N)

    xf = x.reshape(C, N).T
    qkv = _run_a(xf, qkv_w.T)

    U = jnp.asarray(np.triu(np.ones((128, 128), np.float32)))
    L = jnp.asarray(np.tril(np.ones((32, 32), np.float32), -1))
    idx = _run_s(scores.reshape(32, 128), U, L)

    kv = qkv[:, C:]
    kv_g = _run_sc(idx.reshape(512), kv)

    q = qkv[:, :C]
    qh = q.reshape(N, 32, 8).transpose(1, 0, 2)
    kh = kv_g[:, :C].reshape(512, 32, 8).transpose(1, 0, 2)
    vh = kv_g[:, C:].reshape(512, 32, 8).transpose(1, 0, 2)

    oh = _run_c(qh, kh, vh)
    att = oh.transpose(1, 0, 2).reshape(N, C)

    v_full = qkv[:, 2 * C:]
    vpad = jnp.pad(v_full.reshape(D, H, W, C), ((1, 1), (1, 1), (1, 1), (0, 0)))
    w27 = dw1_w.reshape(C, 27).T
    dwout = _run_d(vpad, w27, dw1_b.reshape(1, C)).reshape(N, C)

    bias = (proj_b + dw2_b).reshape(1, C)
    out = _run_e(att, dwout, proj_w.T, dw2_w.reshape(C, C).T, bias)
    return out.T.reshape(B, C, D, H, W)
